# baseline (speedup 1.0000x reference)
import jax
import jax.numpy as jnp
from jax import lax
from jax.experimental import pallas as pl
from jax.experimental.pallas import tpu as pltpu
from jax.experimental.pallas import tpu_sc as plsc

N = 10000
E = 160000
D = 128
R = 50
BS = 2
NEG = 32

NC = 2
NS = 16
EPT = E // NS
SP_ROWS = 5120
HALF = 5056
ZROWS = 64
MATCH_CAP = EPT + 32


def _mesh():
    return plsc.VectorSubcoreMesh(
        core_axis_name="c", subcore_axis_name="s", num_cores=NC, num_subcores=NS
    )


def _sc_layer1(src_h, et_h, dst_h, h0_h, rel0_h, agg1_h,
               src_v, et_v, dst_v, h0_v, zrow, rowbuf, matched, agg_sp,
               sem):
    c = lax.axis_index("c")
    s = lax.axis_index("s")
    base = s * EPT
    pltpu.sync_copy(src_h.at[pl.ds(base, EPT)], src_v)
    pltpu.sync_copy(et_h.at[pl.ds(base, EPT)], et_v)
    pltpu.sync_copy(dst_h.at[pl.ds(base, EPT)], dst_v)
    pltpu.sync_copy(h0_h, h0_v)

    z16 = jnp.zeros((16,), jnp.float32)

    def zfill(r, _):
        for j in range(8):
            zrow[r, pl.ds(j * 16, 16)] = z16
        return 0

    lax.fori_loop(0, ZROWS, zfill, 0)

    hv = h0_v[...]
    h0c = jnp.where(c == 0, hv[0], hv[1])
    lane = lax.iota(jnp.int32, 16)

    def scan(i, m):
        sv = src_v[pl.ds(i * 16, 16)]
        msk = sv == h0c
        cnt = plsc.all_reduce_population_count(msk)[0]

        @pl.when(cnt > 0)
        def _():
            keys = jnp.where(msk, lane, 16 + lane)
            _, vals = plsc.sort_key_val(keys, lane + i * 16)
            matched[pl.ds(m, 16)] = vals

        return m + cnt

    m_total = lax.fori_loop(0, EPT // 16, scan, jnp.int32(0), unroll=4)
    matched[pl.ds(m_total, 16)] = jnp.zeros((16,), jnp.int32)

    dummy = jnp.int32(HALF) + s
    nblk = (m_total + 15) // 16
    rows_per_tile = SP_ROWS // NS

    for h, (half_lo, half_n, chunk) in enumerate(
            (((0, HALF, 312)), (HALF, N - HALF, 304))):

        def zchunk(k, _):
            pltpu.sync_copy(
                zrow, agg_sp.at[pl.ds(s * rows_per_tile + k * ZROWS, ZROWS), :])
            return 0

        lax.fori_loop(0, rows_per_tile // ZROWS, zchunk, 0)
        plsc.subcore_barrier()

        def process(i, _):
            k = i * 16
            idxs = matched[pl.ds(k, 16)]
            ets = plsc.load_gather(et_v, [idxs])
            dsts = plsc.load_gather(dst_v, [idxs])
            in_range = ((lane + k) < m_total) & (dsts >= half_lo) \
                & (dsts < half_lo + half_n)
            rows16 = jnp.where(in_range, dsts - half_lo, dummy)
            pltpu.async_copy(rel0_h.at[c].at[ets], rowbuf, sem).wait()
            pltpu.sync_copy(rowbuf, agg_sp.at[rows16], add=True)
            return 0

        lax.fori_loop(0, nblk, process, 0)
        plsc.subcore_barrier()

        r0 = s * chunk
        pltpu.sync_copy(agg_sp.at[pl.ds(r0, chunk), :],
                        agg1_h.at[c].at[pl.ds(half_lo + r0, chunk), :])
        tail = NS * chunk
        tail_n = half_n - tail

        @pl.when(s == h)
        def _():
            pltpu.sync_copy(agg_sp.at[pl.ds(tail, tail_n), :],
                            agg1_h.at[c].at[pl.ds(half_lo + tail, tail_n), :])

        plsc.subcore_barrier()


def _call_sc_layer1(src, et, dst, h0p, rel0):
    return pl.kernel(
        _sc_layer1,
        out_type=jax.ShapeDtypeStruct((BS, N, D), jnp.float32),
        mesh=_mesh(),
        compiler_params=pltpu.CompilerParams(needs_layout_passes=False),
        scratch_types=[
            pltpu.VMEM((EPT,), jnp.int32),
            pltpu.VMEM((EPT,), jnp.int32),
            pltpu.VMEM((EPT,), jnp.int32),
            pltpu.VMEM((16,), jnp.int32),
            pltpu.VMEM((ZROWS, D), jnp.float32),
            pltpu.VMEM((16, D), jnp.float32),
            pltpu.VMEM((MATCH_CAP,), jnp.int32),
            pltpu.VMEM_SHARED((SP_ROWS + 16, D), jnp.float32),
            pltpu.SemaphoreType.DMA,
        ],
        name="sc_layer1_msg",
    )(src, et, dst, h0p, rel0)


SLOT_ROWS = NEG + NS
BM_WORDS = 320


def _sc_layer2(dst_h, src_h, et_h, t_h, rel1_h, s1_h, agg2_h, s1t_h,
               dst_v, src_v, et_v, t_v, xbuf, rbuf, zrow, outbuf, matched,
               bitmap_v, agg_sp, sem):
    c = lax.axis_index("c")
    s = lax.axis_index("s")
    base = s * EPT
    pltpu.sync_copy(dst_h.at[pl.ds(base, EPT)], dst_v)
    pltpu.sync_copy(src_h.at[pl.ds(base, EPT)], src_v)
    pltpu.sync_copy(et_h.at[pl.ds(base, EPT)], et_v)
    pltpu.sync_copy(t_h, t_v)

    z16 = jnp.zeros((16,), jnp.float32)

    def zfill(r, _):
        for j in range(8):
            zrow[r, pl.ds(j * 16, 16)] = z16
        return 0

    lax.fori_loop(0, 3, zfill, 0)

    @pl.when(s < 3)
    def _():
        pltpu.sync_copy(zrow, agg_sp.at[pl.ds(s * 16, 16), :])

    t_lo = t_v[pl.ds(c * NEG, 16)]
    t_hi = t_v[pl.ds(c * NEG + 16, 16)]
    tj = [t_lo[j] for j in range(16)] + [t_hi[j] for j in range(16)]
    lane = lax.iota(jnp.int32, 16)

    plsc.subcore_barrier()

    for w in range(BM_WORDS // 16):
        wid = lane + w * 16
        bits = jnp.zeros((16,), jnp.int32)
        for j in range(NEG):
            hit = wid == lax.shift_right_logical(tj[j], 5)
            bits = bits | jnp.where(
                hit, lax.shift_left(jnp.int32(1), tj[j] & 31), 0)
        bitmap_v[pl.ds(w * 16, 16)] = bits

    def scan(i, m):
        dv = dst_v[pl.ds(i * 16, 16)]
        words = plsc.load_gather(bitmap_v, [lax.shift_right_logical(dv, 5)])
        anym = (lax.shift_right_logical(words, dv & 31) & 1) != 0
        cnt = plsc.all_reduce_population_count(anym)[0]

        @pl.when(cnt > 0)
        def _():
            keys = jnp.where(anym, lane, 16 + lane)
            _, vals = plsc.sort_key_val(keys, lane + i * 16)
            matched[pl.ds(m, 16)] = vals

        return m + cnt

    m_total = lax.fori_loop(0, EPT // 16, scan, jnp.int32(0), unroll=4)
    matched[pl.ds(m_total, 16)] = jnp.zeros((16,), jnp.int32)

    dummy_slot = jnp.int32(NEG) + s

    def process(i, _):
        k = i * 16
        idxs = matched[pl.ds(k, 16)]
        srcs = plsc.load_gather(src_v, [idxs])
        ets = plsc.load_gather(et_v, [idxs])
        dsts = plsc.load_gather(dst_v, [idxs])
        valid = (lane + k) < m_total
        dsts = jnp.where(valid, dsts, jnp.int32(-1))
        slot = jnp.full((16,), 1, jnp.int32) * dummy_slot
        for j in range(NEG - 1, -1, -1):
            slot = jnp.where(dsts == tj[j], jnp.int32(j), slot)
        pltpu.async_copy(s1_h.at[c].at[srcs], xbuf, sem).wait()
        pltpu.async_copy(rel1_h.at[c].at[ets], rbuf, sem).wait()
        for r in range(16):
            for j in range(8):
                xbuf[r, pl.ds(j * 16, 16)] = (
                    xbuf[r, pl.ds(j * 16, 16)] * rbuf[r, pl.ds(j * 16, 16)]
                )
        pltpu.sync_copy(xbuf, agg_sp.at[slot], add=True)
        return 0

    nblk = (m_total + 15) // 16
    lax.fori_loop(0, nblk, process, 0)
    plsc.subcore_barrier()

    fo_lo = lane
    fo_hi = lane + 16
    for j in range(NEG - 1, -1, -1):
        fo_lo = jnp.where(t_lo == tj[j], jnp.int32(j), fo_lo)
        fo_hi = jnp.where(t_hi == tj[j], jnp.int32(j), fo_hi)

    @pl.when(s == 0)
    def _():
        pltpu.async_copy(agg_sp.at[fo_lo], outbuf, sem).wait()
        pltpu.sync_copy(outbuf, agg2_h.at[c].at[pl.ds(0, 16), :])

    @pl.when(s == 1)
    def _():
        pltpu.async_copy(agg_sp.at[fo_hi], outbuf, sem).wait()
        pltpu.sync_copy(outbuf, agg2_h.at[c].at[pl.ds(16, 16), :])

    @pl.when(s == 2)
    def _():
        pltpu.async_copy(s1_h.at[c].at[t_lo], xbuf, sem).wait()
        pltpu.sync_copy(xbuf, s1t_h.at[c].at[pl.ds(0, 16), :])

    @pl.when(s == 3)
    def _():
        pltpu.async_copy(s1_h.at[c].at[t_hi], xbuf, sem).wait()
        pltpu.sync_copy(xbuf, s1t_h.at[c].at[pl.ds(16, 16), :])


def _call_sc_layer2(dst, src, et, tp, rel1, state1):
    return pl.kernel(
        _sc_layer2,
        out_type=[
            jax.ShapeDtypeStruct((BS, NEG, D), jnp.float32),
            jax.ShapeDtypeStruct((BS, NEG, D), jnp.float32),
        ],
        mesh=_mesh(),
        compiler_params=pltpu.CompilerParams(needs_layout_passes=False),
        scratch_types=[
            pltpu.VMEM((EPT,), jnp.int32),
            pltpu.VMEM((EPT,), jnp.int32),
            pltpu.VMEM((EPT,), jnp.int32),
            pltpu.VMEM((BS * NEG,), jnp.int32),
            pltpu.VMEM((16, D), jnp.float32),
            pltpu.VMEM((16, D), jnp.float32),
            pltpu.VMEM((16, D), jnp.float32),
            pltpu.VMEM((16, D), jnp.float32),
            pltpu.VMEM((MATCH_CAP,), jnp.int32),
            pltpu.VMEM((BM_WORDS,), jnp.int32),
            pltpu.VMEM_SHARED((SLOT_ROWS, D), jnp.float32),
            pltpu.SemaphoreType.DMA,
        ],
        name="sc_layer2_msg",
    )(dst, src, et, tp, rel1, state1)


def _tc_proj(r0_ref, rel_ref, wr0_ref, wr1_ref, rel0_ref, rel1_ref, q_ref):
    rr = rel_ref[...]
    rel0_ref[...] = jnp.dot(rr, wr0_ref[...], preferred_element_type=jnp.float32)
    rel1_ref[...] = jnp.dot(rr, wr1_ref[...], preferred_element_type=jnp.float32)
    q_ref[...] = jnp.zeros((8, D), jnp.float32)
    for b in range(BS):
        idx = b * R + r0_ref[b]
        q_ref[pl.ds(b, 1), :] = rel_ref[pl.ds(idx, 1), :]


def _call_tc_proj(rel_flat, wr0, wr1, r0):
    return pl.pallas_call(
        _tc_proj,
        out_shape=[
            jax.ShapeDtypeStruct((BS * R, D), jnp.float32),
            jax.ShapeDtypeStruct((BS * R, D), jnp.float32),
            jax.ShapeDtypeStruct((8, D), jnp.float32),
        ],
        in_specs=[
            pl.BlockSpec(memory_space=pltpu.SMEM),
            pl.BlockSpec(),
            pl.BlockSpec(),
            pl.BlockSpec(),
        ],
        name="tc_proj",
    )(r0, rel_flat, wr0, wr1)


BN = 2000


def _tc_dense1(h0_ref, agg_ref, q_ref, wx_ref, wa_ref, b_ref, g_ref, bb_ref,
               out_ref):
    b = pl.program_id(0)
    i = pl.program_id(1)
    h0b = h0_ref[b]
    rows = lax.broadcasted_iota(jnp.int32, (BN, 1), 0) + i * BN
    onehot = (rows == h0b).astype(jnp.float32)
    qrow = q_ref[pl.ds(b, 1), :]
    x = onehot * qrow
    a = agg_ref[0] * qrow + x
    qwx = jnp.dot(qrow, wx_ref[...], preferred_element_type=jnp.float32)
    z = (jnp.dot(a, wa_ref[...], preferred_element_type=jnp.float32)
         + onehot * qwx + b_ref[...])
    mu = jnp.mean(z, axis=-1, keepdims=True)
    var = jnp.mean((z - mu) * (z - mu), axis=-1, keepdims=True)
    ln = (z - mu) * lax.rsqrt(var + 1e-5) * g_ref[...] + bb_ref[...]
    out_ref[0] = jnp.maximum(ln, 0.0) + x


def _call_tc_dense1(agg1, query, h0, wx, wa, brow, grow, bbrow):
    grid = (BS, N // BN)
    return pl.pallas_call(
        _tc_dense1,
        grid=grid,
        out_shape=jax.ShapeDtypeStruct((BS, N, D), jnp.float32),
        in_specs=[
            pl.BlockSpec(memory_space=pltpu.SMEM),
            pl.BlockSpec((1, BN, D), lambda b, i: (b, i, 0)),
            pl.BlockSpec((BS, D), lambda b, i: (0, 0)),
            pl.BlockSpec((D, D), lambda b, i: (0, 0)),
            pl.BlockSpec((D, D), lambda b, i: (0, 0)),
            pl.BlockSpec((1, D), lambda b, i: (0, 0)),
            pl.BlockSpec((1, D), lambda b, i: (0, 0)),
            pl.BlockSpec((1, D), lambda b, i: (0, 0)),
        ],
        out_specs=pl.BlockSpec((1, BN, D), lambda b, i: (b, i, 0)),
        name="tc_dense1",
    )(h0, agg1, query, wx, wa, brow, grow, bbrow)


def _tc_head(b2_ref, x_ref, a_ref, qb_ref, bm_ref, wx_ref, wa_ref, b_ref,
             g_ref, bb_ref, w1_ref, b1_ref, w2_ref, out_ref):
    x = x_ref[...]
    a = a_ref[...] + bm_ref[...] * qb_ref[...]
    z = (jnp.dot(x, wx_ref[...], preferred_element_type=jnp.float32)
         + jnp.dot(a, wa_ref[...], preferred_element_type=jnp.float32)
         + b_ref[...])
    mu = jnp.mean(z, axis=-1, keepdims=True)
    var = jnp.mean((z - mu) * (z - mu), axis=-1, keepdims=True)
    ln = (z - mu) * lax.rsqrt(var + 1e-5) * g_ref[...] + bb_ref[...]
    h = jnp.maximum(ln, 0.0) + x
    feat = jnp.concatenate([h, qb_ref[...]], axis=1)
    hm = jnp.maximum(
        jnp.dot(feat, w1_ref[...], preferred_element_type=jnp.float32)
        + b1_ref[...], 0.0)
    sc = jnp.sum(hm * w2_ref[...], axis=1, keepdims=True) + b2_ref[0]
    out_ref[...] = sc


def _call_tc_head(s1t, agg2, qb, bmask, wx, wa, brow, grow, bbrow,
                  mlp_w1, mlp_b1row, w2row, b2):
    return pl.pallas_call(
        _tc_head,
        out_shape=jax.ShapeDtypeStruct((BS * NEG, 1), jnp.float32),
        in_specs=[pl.BlockSpec(memory_space=pltpu.SMEM)] + [pl.BlockSpec()] * 12,
        name="tc_head",
    )(b2, s1t, agg2, qb, bmask, wx, wa, brow, grow, bbrow, mlp_w1, mlp_b1row,
      w2row)


def kernel(relation_representations, batch, edge_index, edge_type,
           Wr0, W0, b0, g0, bb0, Wr1, W1, b1, g1, bb1,
           mlp_W1, mlp_b1, mlp_W2, mlp_b2):
    h0 = batch[:, 0, 0].astype(jnp.int32)
    t_flat = batch[:, :, 1].reshape(BS * NEG).astype(jnp.int32)
    r0 = batch[:, 0, 2].astype(jnp.int32)
    src = edge_index[0]
    dst = edge_index[1]

    rel_flat = relation_representations.reshape(BS * R, D)
    rel0f, rel1f, q8 = _call_tc_proj(rel_flat, Wr0, Wr1, r0)
    query = q8[:BS]
    rel0 = rel0f.reshape(BS, R, D)
    rel1 = rel1f.reshape(BS, R, D)

    h0p = jnp.zeros((16,), jnp.int32).at[:BS].set(h0)
    agg1 = _call_sc_layer1(src, edge_type, dst, h0p, rel0)

    state1 = _call_tc_dense1(
        agg1, query, h0, W0[:D], W0[D:], b0.reshape(1, D), g0.reshape(1, D),
        bb0.reshape(1, D))

    agg2, s1t = _call_sc_layer2(dst, src, edge_type, t_flat, rel1, state1)

    qb = jnp.repeat(query, NEG, axis=0)
    bmask = (t_flat == jnp.repeat(h0, NEG)).astype(jnp.float32)[:, None]
    score = _call_tc_head(
        s1t.reshape(BS * NEG, D), agg2.reshape(BS * NEG, D), qb, bmask,
        W1[:D], W1[D:], b1.reshape(1, D), g1.reshape(1, D), bb1.reshape(1, D),
        mlp_W1, mlp_b1.reshape(1, 2 * D), mlp_W2.reshape(1, 2 * D), mlp_b2)
    return score.reshape(BS, NEG)


# speedup vs baseline: 1.0957x; 1.0957x over previous
import jax
import jax.numpy as jnp
from jax import lax
from jax.experimental import pallas as pl
from jax.experimental.pallas import tpu as pltpu
from jax.experimental.pallas import tpu_sc as plsc

N = 10000
E = 160000
D = 128
R = 50
BS = 2
NEG = 32

NC = 2
NS = 16
EPT = E // NS
SP_ROWS = 5120
HALF = 5056
ZROWS = 64
MATCH_CAP = EPT + 32


def _mesh():
    return plsc.VectorSubcoreMesh(
        core_axis_name="c", subcore_axis_name="s", num_cores=NC, num_subcores=NS
    )


def _sc_layer1(src_h, et_h, dst_h, h0_h, rel0_h, agg1_h,
               src_v, et_v, dst_v, h0_v, zrow, rowbuf, matched, agg_sp,
               sem):
    c = lax.axis_index("c")
    s = lax.axis_index("s")
    base = s * EPT
    cps = [pltpu.async_copy(src_h.at[pl.ds(base, EPT)], src_v, sem),
           pltpu.async_copy(et_h.at[pl.ds(base, EPT)], et_v, sem),
           pltpu.async_copy(dst_h.at[pl.ds(base, EPT)], dst_v, sem),
           pltpu.async_copy(h0_h, h0_v, sem)]
    for cp in cps:
        cp.wait()

    z16 = jnp.zeros((16,), jnp.float32)

    def zfill(r, _):
        for j in range(8):
            zrow[r, pl.ds(j * 16, 16)] = z16
        return 0

    lax.fori_loop(0, ZROWS, zfill, 0)

    hv = h0_v[...]
    h0c = jnp.where(c == 0, hv[0], hv[1])
    lane = lax.iota(jnp.int32, 16)

    def scan(i, m):
        sv = src_v[pl.ds(i * 16, 16)]
        msk = sv == h0c
        cnt = plsc.all_reduce_population_count(msk)[0]
        keys = jnp.where(msk, lane, 16 + lane)
        _, vals = plsc.sort_key_val(keys, lane + i * 16)
        matched[pl.ds(m, 16)] = vals
        return m + cnt

    m_total = lax.fori_loop(0, EPT // 16, scan, jnp.int32(0))
    matched[pl.ds(m_total, 16)] = jnp.zeros((16,), jnp.int32)

    dummy = jnp.int32(HALF) + s
    nblk = (m_total + 15) // 16
    rows_per_tile = SP_ROWS // NS

    for h, (half_lo, half_n, chunk) in enumerate(
            (((0, HALF, 312)), (HALF, N - HALF, 304))):

        zcps = [
            pltpu.async_copy(
                zrow, agg_sp.at[pl.ds(s * rows_per_tile + k * ZROWS, ZROWS), :],
                sem)
            for k in range(rows_per_tile // ZROWS)]
        for cp in zcps:
            cp.wait()
        plsc.subcore_barrier()

        def process(i, _):
            k = i * 16
            idxs = matched[pl.ds(k, 16)]
            ets = plsc.load_gather(et_v, [idxs])
            dsts = plsc.load_gather(dst_v, [idxs])
            in_range = ((lane + k) < m_total) & (dsts >= half_lo) \
                & (dsts < half_lo + half_n)
            rows16 = jnp.where(in_range, dsts - half_lo, dummy)
            pltpu.async_copy(rel0_h.at[c].at[ets], rowbuf, sem).wait()
            pltpu.sync_copy(rowbuf, agg_sp.at[rows16], add=True)
            return 0

        lax.fori_loop(0, nblk, process, 0)
        plsc.subcore_barrier()

        r0 = s * chunk
        pltpu.sync_copy(agg_sp.at[pl.ds(r0, chunk), :],
                        agg1_h.at[c].at[pl.ds(half_lo + r0, chunk), :])
        tail = NS * chunk
        tail_n = half_n - tail

        @pl.when(s == h)
        def _():
            pltpu.sync_copy(agg_sp.at[pl.ds(tail, tail_n), :],
                            agg1_h.at[c].at[pl.ds(half_lo + tail, tail_n), :])

        plsc.subcore_barrier()


def _call_sc_layer1(src, et, dst, h0p, rel0):
    return pl.kernel(
        _sc_layer1,
        out_type=jax.ShapeDtypeStruct((BS, N, D), jnp.float32),
        mesh=_mesh(),
        compiler_params=pltpu.CompilerParams(needs_layout_passes=False),
        scratch_types=[
            pltpu.VMEM((EPT,), jnp.int32),
            pltpu.VMEM((EPT,), jnp.int32),
            pltpu.VMEM((EPT,), jnp.int32),
            pltpu.VMEM((16,), jnp.int32),
            pltpu.VMEM((ZROWS, D), jnp.float32),
            pltpu.VMEM((16, D), jnp.float32),
            pltpu.VMEM((MATCH_CAP,), jnp.int32),
            pltpu.VMEM_SHARED((SP_ROWS + 16, D), jnp.float32),
            pltpu.SemaphoreType.DMA,
        ],
        name="sc_layer1_msg",
    )(src, et, dst, h0p, rel0)


SLOT_ROWS = NEG + NS
BM_WORDS = 320


def _sc_layer2(dst_h, src_h, et_h, t_h, rel1_h, s1_h, agg2_h, s1t_h,
               dst_v, src_v, et_v, t_v, xbuf, rbuf, zrow, outbuf, matched,
               agg_sp, sem):
    c = lax.axis_index("c")
    s = lax.axis_index("s")
    base = s * EPT
    cps = [pltpu.async_copy(dst_h.at[pl.ds(base, EPT)], dst_v, sem),
           pltpu.async_copy(src_h.at[pl.ds(base, EPT)], src_v, sem),
           pltpu.async_copy(et_h.at[pl.ds(base, EPT)], et_v, sem),
           pltpu.async_copy(t_h, t_v, sem)]
    for cp in cps:
        cp.wait()

    z16 = jnp.zeros((16,), jnp.float32)

    def zfill(r, _):
        for j in range(8):
            zrow[r, pl.ds(j * 16, 16)] = z16
        return 0

    lax.fori_loop(0, 3, zfill, 0)

    @pl.when(s < 3)
    def _():
        pltpu.sync_copy(zrow, agg_sp.at[pl.ds(s * 16, 16), :])

    t_lo = t_v[pl.ds(c * NEG, 16)]
    t_hi = t_v[pl.ds(c * NEG + 16, 16)]
    tj = [t_lo[j] for j in range(16)] + [t_hi[j] for j in range(16)]
    lane = lax.iota(jnp.int32, 16)

    plsc.subcore_barrier()

    def scan(i, m):
        dv = dst_v[pl.ds(i * 16, 16)]
        anym = dv == tj[0]
        for j in range(1, NEG):
            anym = jnp.logical_or(anym, dv == tj[j])
        cnt = plsc.all_reduce_population_count(anym)[0]
        keys = jnp.where(anym, lane, 16 + lane)
        _, vals = plsc.sort_key_val(keys, lane + i * 16)
        matched[pl.ds(m, 16)] = vals
        return m + cnt

    m_total = lax.fori_loop(0, EPT // 16, scan, jnp.int32(0))
    matched[pl.ds(m_total, 16)] = jnp.zeros((16,), jnp.int32)

    dummy_slot = jnp.int32(NEG) + s

    def process(i, _):
        k = i * 16
        idxs = matched[pl.ds(k, 16)]
        srcs = plsc.load_gather(src_v, [idxs])
        ets = plsc.load_gather(et_v, [idxs])
        dsts = plsc.load_gather(dst_v, [idxs])
        valid = (lane + k) < m_total
        dsts = jnp.where(valid, dsts, jnp.int32(-1))
        slot = jnp.full((16,), 1, jnp.int32) * dummy_slot
        for j in range(NEG - 1, -1, -1):
            slot = jnp.where(dsts == tj[j], jnp.int32(j), slot)
        pltpu.async_copy(s1_h.at[c].at[srcs], xbuf, sem).wait()
        pltpu.async_copy(rel1_h.at[c].at[ets], rbuf, sem).wait()
        for r in range(16):
            for j in range(8):
                xbuf[r, pl.ds(j * 16, 16)] = (
                    xbuf[r, pl.ds(j * 16, 16)] * rbuf[r, pl.ds(j * 16, 16)]
                )
        pltpu.sync_copy(xbuf, agg_sp.at[slot], add=True)
        return 0

    nblk = (m_total + 15) // 16
    lax.fori_loop(0, nblk, process, 0)
    plsc.subcore_barrier()

    fo_lo = lane
    fo_hi = lane + 16
    for j in range(NEG - 1, -1, -1):
        fo_lo = jnp.where(t_lo == tj[j], jnp.int32(j), fo_lo)
        fo_hi = jnp.where(t_hi == tj[j], jnp.int32(j), fo_hi)

    @pl.when(s == 0)
    def _():
        pltpu.async_copy(agg_sp.at[fo_lo], outbuf, sem).wait()
        pltpu.sync_copy(outbuf, agg2_h.at[c].at[pl.ds(0, 16), :])

    @pl.when(s == 1)
    def _():
        pltpu.async_copy(agg_sp.at[fo_hi], outbuf, sem).wait()
        pltpu.sync_copy(outbuf, agg2_h.at[c].at[pl.ds(16, 16), :])

    @pl.when(s == 2)
    def _():
        pltpu.async_copy(s1_h.at[c].at[t_lo], xbuf, sem).wait()
        pltpu.sync_copy(xbuf, s1t_h.at[c].at[pl.ds(0, 16), :])

    @pl.when(s == 3)
    def _():
        pltpu.async_copy(s1_h.at[c].at[t_hi], xbuf, sem).wait()
        pltpu.sync_copy(xbuf, s1t_h.at[c].at[pl.ds(16, 16), :])


def _call_sc_layer2(dst, src, et, tp, rel1, state1):
    return pl.kernel(
        _sc_layer2,
        out_type=[
            jax.ShapeDtypeStruct((BS, NEG, D), jnp.float32),
            jax.ShapeDtypeStruct((BS, NEG, D), jnp.float32),
        ],
        mesh=_mesh(),
        compiler_params=pltpu.CompilerParams(needs_layout_passes=False),
        scratch_types=[
            pltpu.VMEM((EPT,), jnp.int32),
            pltpu.VMEM((EPT,), jnp.int32),
            pltpu.VMEM((EPT,), jnp.int32),
            pltpu.VMEM((BS * NEG,), jnp.int32),
            pltpu.VMEM((16, D), jnp.float32),
            pltpu.VMEM((16, D), jnp.float32),
            pltpu.VMEM((16, D), jnp.float32),
            pltpu.VMEM((16, D), jnp.float32),
            pltpu.VMEM((MATCH_CAP,), jnp.int32),
            pltpu.VMEM_SHARED((SLOT_ROWS, D), jnp.float32),
            pltpu.SemaphoreType.DMA,
        ],
        name="sc_layer2_msg",
    )(dst, src, et, tp, rel1, state1)


def _tc_proj(r0_ref, rel_ref, wr0_ref, wr1_ref, rel0_ref, rel1_ref, q_ref):
    rr = rel_ref[...]
    rel0_ref[...] = jnp.dot(rr, wr0_ref[...], preferred_element_type=jnp.float32)
    rel1_ref[...] = jnp.dot(rr, wr1_ref[...], preferred_element_type=jnp.float32)
    q_ref[...] = jnp.zeros((8, D), jnp.float32)
    for b in range(BS):
        idx = b * R + r0_ref[b]
        q_ref[pl.ds(b, 1), :] = rel_ref[pl.ds(idx, 1), :]


def _call_tc_proj(rel_flat, wr0, wr1, r0):
    return pl.pallas_call(
        _tc_proj,
        out_shape=[
            jax.ShapeDtypeStruct((BS * R, D), jnp.float32),
            jax.ShapeDtypeStruct((BS * R, D), jnp.float32),
            jax.ShapeDtypeStruct((8, D), jnp.float32),
        ],
        in_specs=[
            pl.BlockSpec(memory_space=pltpu.SMEM),
            pl.BlockSpec(),
            pl.BlockSpec(),
            pl.BlockSpec(),
        ],
        name="tc_proj",
    )(r0, rel_flat, wr0, wr1)


BN = 2000


def _tc_dense1(h0_ref, agg_ref, q_ref, wx_ref, wa_ref, b_ref, g_ref, bb_ref,
               out_ref):
    b = pl.program_id(0)
    i = pl.program_id(1)
    h0b = h0_ref[b]
    rows = lax.broadcasted_iota(jnp.int32, (BN, 1), 0) + i * BN
    onehot = (rows == h0b).astype(jnp.float32)
    qrow = q_ref[pl.ds(b, 1), :]
    x = onehot * qrow
    a = agg_ref[0] * qrow + x
    qwx = jnp.dot(qrow, wx_ref[...], preferred_element_type=jnp.float32)
    z = (jnp.dot(a, wa_ref[...], preferred_element_type=jnp.float32)
         + onehot * qwx + b_ref[...])
    mu = jnp.mean(z, axis=-1, keepdims=True)
    var = jnp.mean((z - mu) * (z - mu), axis=-1, keepdims=True)
    ln = (z - mu) * lax.rsqrt(var + 1e-5) * g_ref[...] + bb_ref[...]
    out_ref[0] = jnp.maximum(ln, 0.0) + x


def _call_tc_dense1(agg1, query, h0, wx, wa, brow, grow, bbrow):
    grid = (BS, N // BN)
    return pl.pallas_call(
        _tc_dense1,
        grid=grid,
        out_shape=jax.ShapeDtypeStruct((BS, N, D), jnp.float32),
        in_specs=[
            pl.BlockSpec(memory_space=pltpu.SMEM),
            pl.BlockSpec((1, BN, D), lambda b, i: (b, i, 0)),
            pl.BlockSpec((BS, D), lambda b, i: (0, 0)),
            pl.BlockSpec((D, D), lambda b, i: (0, 0)),
            pl.BlockSpec((D, D), lambda b, i: (0, 0)),
            pl.BlockSpec((1, D), lambda b, i: (0, 0)),
            pl.BlockSpec((1, D), lambda b, i: (0, 0)),
            pl.BlockSpec((1, D), lambda b, i: (0, 0)),
        ],
        out_specs=pl.BlockSpec((1, BN, D), lambda b, i: (b, i, 0)),
        name="tc_dense1",
    )(h0, agg1, query, wx, wa, brow, grow, bbrow)


def _tc_head(b2_ref, x_ref, a_ref, qb_ref, bm_ref, wx_ref, wa_ref, b_ref,
             g_ref, bb_ref, w1_ref, b1_ref, w2_ref, out_ref):
    x = x_ref[...]
    a = a_ref[...] + bm_ref[...] * qb_ref[...]
    z = (jnp.dot(x, wx_ref[...], preferred_element_type=jnp.float32)
         + jnp.dot(a, wa_ref[...], preferred_element_type=jnp.float32)
         + b_ref[...])
    mu = jnp.mean(z, axis=-1, keepdims=True)
    var = jnp.mean((z - mu) * (z - mu), axis=-1, keepdims=True)
    ln = (z - mu) * lax.rsqrt(var + 1e-5) * g_ref[...] + bb_ref[...]
    h = jnp.maximum(ln, 0.0) + x
    feat = jnp.concatenate([h, qb_ref[...]], axis=1)
    hm = jnp.maximum(
        jnp.dot(feat, w1_ref[...], preferred_element_type=jnp.float32)
        + b1_ref[...], 0.0)
    sc = jnp.sum(hm * w2_ref[...], axis=1, keepdims=True) + b2_ref[0]
    out_ref[...] = sc


def _call_tc_head(s1t, agg2, qb, bmask, wx, wa, brow, grow, bbrow,
                  mlp_w1, mlp_b1row, w2row, b2):
    return pl.pallas_call(
        _tc_head,
        out_shape=jax.ShapeDtypeStruct((BS * NEG, 1), jnp.float32),
        in_specs=[pl.BlockSpec(memory_space=pltpu.SMEM)] + [pl.BlockSpec()] * 12,
        name="tc_head",
    )(b2, s1t, agg2, qb, bmask, wx, wa, brow, grow, bbrow, mlp_w1, mlp_b1row,
      w2row)


def kernel(relation_representations, batch, edge_index, edge_type,
           Wr0, W0, b0, g0, bb0, Wr1, W1, b1, g1, bb1,
           mlp_W1, mlp_b1, mlp_W2, mlp_b2):
    h0 = batch[:, 0, 0].astype(jnp.int32)
    t_flat = batch[:, :, 1].reshape(BS * NEG).astype(jnp.int32)
    r0 = batch[:, 0, 2].astype(jnp.int32)
    src = edge_index[0]
    dst = edge_index[1]

    rel_flat = relation_representations.reshape(BS * R, D)
    rel0f, rel1f, q8 = _call_tc_proj(rel_flat, Wr0, Wr1, r0)
    query = q8[:BS]
    rel0 = rel0f.reshape(BS, R, D)
    rel1 = rel1f.reshape(BS, R, D)

    h0p = jnp.zeros((16,), jnp.int32).at[:BS].set(h0)
    agg1 = _call_sc_layer1(src, edge_type, dst, h0p, rel0)

    state1 = _call_tc_dense1(
        agg1, query, h0, W0[:D], W0[D:], b0.reshape(1, D), g0.reshape(1, D),
        bb0.reshape(1, D))

    agg2, s1t = _call_sc_layer2(dst, src, edge_type, t_flat, rel1, state1)

    qb = jnp.repeat(query, NEG, axis=0)
    bmask = (t_flat == jnp.repeat(h0, NEG)).astype(jnp.float32)[:, None]
    score = _call_tc_head(
        s1t.reshape(BS * NEG, D), agg2.reshape(BS * NEG, D), qb, bmask,
        W1[:D], W1[D:], b1.reshape(1, D), g1.reshape(1, D), bb1.reshape(1, D),
        mlp_W1, mlp_b1.reshape(1, 2 * D), mlp_W2.reshape(1, 2 * D), mlp_b2)
    return score.reshape(BS, NEG)


# speedup vs baseline: 1.1099x; 1.0129x over previous
import jax
import jax.numpy as jnp
from jax import lax
from jax.experimental import pallas as pl
from jax.experimental.pallas import tpu as pltpu
from jax.experimental.pallas import tpu_sc as plsc

N = 10000
E = 160000
D = 128
R = 50
BS = 2
NEG = 32

NC = 2
NS = 16
EPT = E // NS
SP_ROWS = 5120
HALF = 5056
ZROWS = 64
MATCH_CAP = EPT + 32


def _mesh():
    return plsc.VectorSubcoreMesh(
        core_axis_name="c", subcore_axis_name="s", num_cores=NC, num_subcores=NS
    )


def _sc_layer1(src_h, et_h, dst_h, h0_h, rel0_h, agg1_h,
               src_v, et_v, dst_v, h0_v, zrow, rowbuf, matched, agg_sp,
               sem):
    c = lax.axis_index("c")
    s = lax.axis_index("s")
    base = s * EPT
    cps = [pltpu.async_copy(src_h.at[pl.ds(base, EPT)], src_v, sem),
           pltpu.async_copy(et_h.at[pl.ds(base, EPT)], et_v, sem),
           pltpu.async_copy(dst_h.at[pl.ds(base, EPT)], dst_v, sem),
           pltpu.async_copy(h0_h, h0_v, sem)]
    for cp in cps:
        cp.wait()

    z16 = jnp.zeros((16,), jnp.float32)

    def zfill(r, _):
        for j in range(8):
            zrow[r, pl.ds(j * 16, 16)] = z16
        return 0

    lax.fori_loop(0, ZROWS, zfill, 0)

    hv = h0_v[...]
    h0c = jnp.where(c == 0, hv[0], hv[1])
    lane = lax.iota(jnp.int32, 16)

    def scan(i, m):
        sv = src_v[pl.ds(i * 16, 16)]
        msk = sv == h0c
        cnt = plsc.all_reduce_population_count(msk)[0]
        keys = jnp.where(msk, lane, 16 + lane)
        _, vals = plsc.sort_key_val(keys, lane + i * 16)
        matched[pl.ds(m, 16)] = vals
        return m + cnt

    m_total = lax.fori_loop(0, EPT // 16, scan, jnp.int32(0))
    matched[pl.ds(m_total, 16)] = jnp.zeros((16,), jnp.int32)

    dummy = jnp.int32(HALF) + s
    nblk = (m_total + 15) // 16
    rows_per_tile = SP_ROWS // NS

    for h, (half_lo, half_n, chunk) in enumerate(
            (((0, HALF, 312)), (HALF, N - HALF, 304))):

        zcps = [
            pltpu.async_copy(
                zrow, agg_sp.at[pl.ds(s * rows_per_tile + k * ZROWS, ZROWS), :],
                sem)
            for k in range(rows_per_tile // ZROWS)]
        for cp in zcps:
            cp.wait()
        plsc.subcore_barrier()

        def process(i, _):
            k = i * 16
            idxs = matched[pl.ds(k, 16)]
            ets = plsc.load_gather(et_v, [idxs])
            dsts = plsc.load_gather(dst_v, [idxs])
            in_range = ((lane + k) < m_total) & (dsts >= half_lo) \
                & (dsts < half_lo + half_n)
            rows16 = jnp.where(in_range, dsts - half_lo, dummy)
            pltpu.async_copy(rel0_h.at[c].at[ets], rowbuf, sem).wait()
            pltpu.sync_copy(rowbuf, agg_sp.at[rows16], add=True)
            return 0

        lax.fori_loop(0, nblk, process, 0)
        plsc.subcore_barrier()

        r0 = s * chunk
        pltpu.sync_copy(agg_sp.at[pl.ds(r0, chunk), :],
                        agg1_h.at[c].at[pl.ds(half_lo + r0, chunk), :])
        tail = NS * chunk
        tail_n = half_n - tail

        @pl.when(s == h)
        def _():
            pltpu.sync_copy(agg_sp.at[pl.ds(tail, tail_n), :],
                            agg1_h.at[c].at[pl.ds(half_lo + tail, tail_n), :])

        plsc.subcore_barrier()


def _call_sc_layer1(src, et, dst, h0p, rel0):
    return pl.kernel(
        _sc_layer1,
        out_type=jax.ShapeDtypeStruct((BS, N, D), jnp.float32),
        mesh=_mesh(),
        compiler_params=pltpu.CompilerParams(needs_layout_passes=False),
        scratch_types=[
            pltpu.VMEM((EPT,), jnp.int32),
            pltpu.VMEM((EPT,), jnp.int32),
            pltpu.VMEM((EPT,), jnp.int32),
            pltpu.VMEM((16,), jnp.int32),
            pltpu.VMEM((ZROWS, D), jnp.float32),
            pltpu.VMEM((16, D), jnp.float32),
            pltpu.VMEM((MATCH_CAP,), jnp.int32),
            pltpu.VMEM_SHARED((SP_ROWS + 16, D), jnp.float32),
            pltpu.SemaphoreType.DMA,
        ],
        name="sc_layer1_msg",
    )(src, et, dst, h0p, rel0)


SLOT_ROWS = NEG + NS
BM_WORDS = 320


def _sc_layer2(dst_h, src_h, et_h, t_h, rel1_h, s1_h, agg2_h, s1t_h,
               dst_v, src_v, et_v, t_v, xbuf, rbuf, zrow, outbuf, matched,
               agg_sp, sem):
    c = lax.axis_index("c")
    s = lax.axis_index("s")
    base = s * EPT
    cps = [pltpu.async_copy(dst_h.at[pl.ds(base, EPT)], dst_v, sem),
           pltpu.async_copy(src_h.at[pl.ds(base, EPT)], src_v, sem),
           pltpu.async_copy(et_h.at[pl.ds(base, EPT)], et_v, sem),
           pltpu.async_copy(t_h, t_v, sem)]
    for cp in cps:
        cp.wait()

    z16 = jnp.zeros((16,), jnp.float32)

    def zfill(r, _):
        for j in range(8):
            zrow[r, pl.ds(j * 16, 16)] = z16
        return 0

    lax.fori_loop(0, 3, zfill, 0)

    @pl.when(s < 3)
    def _():
        pltpu.sync_copy(zrow, agg_sp.at[pl.ds(s * 16, 16), :])

    t_lo = t_v[pl.ds(c * NEG, 16)]
    t_hi = t_v[pl.ds(c * NEG + 16, 16)]
    tj = [t_lo[j] for j in range(16)] + [t_hi[j] for j in range(16)]
    lane = lax.iota(jnp.int32, 16)

    plsc.subcore_barrier()

    def scan(i, m):
        dv = dst_v[pl.ds(i * 16, 16)]
        anym = dv == tj[0]
        for j in range(1, NEG):
            anym = jnp.logical_or(anym, dv == tj[j])
        cnt = plsc.all_reduce_population_count(anym)[0]
        keys = jnp.where(anym, lane, 16 + lane)
        _, vals = plsc.sort_key_val(keys, lane + i * 16)
        matched[pl.ds(m, 16)] = vals
        return m + cnt

    m_total = lax.fori_loop(0, EPT // 16, scan, jnp.int32(0))
    matched[pl.ds(m_total, 16)] = jnp.zeros((16,), jnp.int32)

    dummy_slot = jnp.int32(NEG) + s

    def process(i, _):
        k = i * 16
        idxs = matched[pl.ds(k, 16)]
        srcs = plsc.load_gather(src_v, [idxs])
        ets = plsc.load_gather(et_v, [idxs])
        dsts = plsc.load_gather(dst_v, [idxs])
        valid = (lane + k) < m_total
        dsts = jnp.where(valid, dsts, jnp.int32(-1))
        slot = jnp.full((16,), 1, jnp.int32) * dummy_slot
        for j in range(NEG - 1, -1, -1):
            slot = jnp.where(dsts == tj[j], jnp.int32(j), slot)
        pltpu.async_copy(s1_h.at[c].at[srcs], xbuf, sem).wait()
        pltpu.async_copy(rel1_h.at[c].at[ets], rbuf, sem).wait()
        for r in range(16):
            for j in range(8):
                xbuf[r, pl.ds(j * 16, 16)] = (
                    xbuf[r, pl.ds(j * 16, 16)] * rbuf[r, pl.ds(j * 16, 16)]
                )
        pltpu.sync_copy(xbuf, agg_sp.at[slot], add=True)
        return 0

    nblk = (m_total + 15) // 16
    lax.fori_loop(0, nblk, process, 0)
    plsc.subcore_barrier()

    fo_lo = lane
    fo_hi = lane + 16
    for j in range(NEG - 1, -1, -1):
        fo_lo = jnp.where(t_lo == tj[j], jnp.int32(j), fo_lo)
        fo_hi = jnp.where(t_hi == tj[j], jnp.int32(j), fo_hi)

    @pl.when(s == 0)
    def _():
        pltpu.async_copy(agg_sp.at[fo_lo], outbuf, sem).wait()
        pltpu.sync_copy(outbuf, agg2_h.at[c].at[pl.ds(0, 16), :])

    @pl.when(s == 1)
    def _():
        pltpu.async_copy(agg_sp.at[fo_hi], outbuf, sem).wait()
        pltpu.sync_copy(outbuf, agg2_h.at[c].at[pl.ds(16, 16), :])

    @pl.when(s == 2)
    def _():
        pltpu.async_copy(s1_h.at[c].at[t_lo], xbuf, sem).wait()
        pltpu.sync_copy(xbuf, s1t_h.at[c].at[pl.ds(0, 16), :])

    @pl.when(s == 3)
    def _():
        pltpu.async_copy(s1_h.at[c].at[t_hi], xbuf, sem).wait()
        pltpu.sync_copy(xbuf, s1t_h.at[c].at[pl.ds(16, 16), :])


def _call_sc_layer2(dst, src, et, tp, rel1, state1):
    return pl.kernel(
        _sc_layer2,
        out_type=[
            jax.ShapeDtypeStruct((BS, NEG, D), jnp.float32),
            jax.ShapeDtypeStruct((BS, NEG, D), jnp.float32),
        ],
        mesh=_mesh(),
        compiler_params=pltpu.CompilerParams(needs_layout_passes=False),
        scratch_types=[
            pltpu.VMEM((EPT,), jnp.int32),
            pltpu.VMEM((EPT,), jnp.int32),
            pltpu.VMEM((EPT,), jnp.int32),
            pltpu.VMEM((BS * NEG,), jnp.int32),
            pltpu.VMEM((16, D), jnp.float32),
            pltpu.VMEM((16, D), jnp.float32),
            pltpu.VMEM((16, D), jnp.float32),
            pltpu.VMEM((16, D), jnp.float32),
            pltpu.VMEM((MATCH_CAP,), jnp.int32),
            pltpu.VMEM_SHARED((SLOT_ROWS, D), jnp.float32),
            pltpu.SemaphoreType.DMA,
        ],
        name="sc_layer2_msg",
    )(dst, src, et, tp, rel1, state1)


def _tc_proj(r0_ref, rel_ref, wr1_ref, rel1_ref, q_ref):
    rr = rel_ref[...]
    rel1_ref[...] = jnp.dot(rr, wr1_ref[...], preferred_element_type=jnp.float32)
    q_ref[...] = jnp.zeros((8, D), jnp.float32)
    for b in range(BS):
        idx = b * R + r0_ref[b]
        q_ref[pl.ds(b, 1), :] = rel_ref[pl.ds(idx, 1), :]


def _call_tc_proj(rel_flat, wr1, r0):
    return pl.pallas_call(
        _tc_proj,
        out_shape=[
            jax.ShapeDtypeStruct((BS * R, D), jnp.float32),
            jax.ShapeDtypeStruct((8, D), jnp.float32),
        ],
        in_specs=[
            pl.BlockSpec(memory_space=pltpu.SMEM),
            pl.BlockSpec(),
            pl.BlockSpec(),
        ],
        name="tc_proj",
    )(r0, rel_flat, wr1)


BN = 2000


def _tc_dense1(h0_ref, agg_ref, q_ref, wr0_ref, wx_ref, wa_ref, b_ref,
               g_ref, bb_ref, out_ref):
    b = pl.program_id(0)
    i = pl.program_id(1)
    h0b = h0_ref[b]
    rows = lax.broadcasted_iota(jnp.int32, (BN, 1), 0) + i * BN
    onehot = (rows == h0b).astype(jnp.float32)
    qrow = q_ref[pl.ds(b, 1), :]
    x = onehot * qrow
    aggp = jnp.dot(agg_ref[0], wr0_ref[...], preferred_element_type=jnp.float32)
    a = aggp * qrow + x
    qwx = jnp.dot(qrow, wx_ref[...], preferred_element_type=jnp.float32)
    z = (jnp.dot(a, wa_ref[...], preferred_element_type=jnp.float32)
         + onehot * qwx + b_ref[...])
    mu = jnp.mean(z, axis=-1, keepdims=True)
    var = jnp.mean((z - mu) * (z - mu), axis=-1, keepdims=True)
    ln = (z - mu) * lax.rsqrt(var + 1e-5) * g_ref[...] + bb_ref[...]
    out_ref[0] = jnp.maximum(ln, 0.0) + x


def _call_tc_dense1(agg1, query, h0, wr0, wx, wa, brow, grow, bbrow):
    grid = (BS, N // BN)
    return pl.pallas_call(
        _tc_dense1,
        grid=grid,
        out_shape=jax.ShapeDtypeStruct((BS, N, D), jnp.float32),
        in_specs=[
            pl.BlockSpec(memory_space=pltpu.SMEM),
            pl.BlockSpec((1, BN, D), lambda b, i: (b, i, 0)),
            pl.BlockSpec((8, D), lambda b, i: (0, 0)),
            pl.BlockSpec((D, D), lambda b, i: (0, 0)),
            pl.BlockSpec((D, D), lambda b, i: (0, 0)),
            pl.BlockSpec((D, D), lambda b, i: (0, 0)),
            pl.BlockSpec((1, D), lambda b, i: (0, 0)),
            pl.BlockSpec((1, D), lambda b, i: (0, 0)),
            pl.BlockSpec((1, D), lambda b, i: (0, 0)),
        ],
        out_specs=pl.BlockSpec((1, BN, D), lambda b, i: (b, i, 0)),
        name="tc_dense1",
    )(h0, agg1, query, wr0, wx, wa, brow, grow, bbrow)


def _tc_head(b2_ref, x_ref, a_ref, qb_ref, bm_ref, wx_ref, wa_ref, b_ref,
             g_ref, bb_ref, w1_ref, b1_ref, w2_ref, out_ref):
    x = x_ref[...]
    a = a_ref[...] + bm_ref[...] * qb_ref[...]
    z = (jnp.dot(x, wx_ref[...], preferred_element_type=jnp.float32)
         + jnp.dot(a, wa_ref[...], preferred_element_type=jnp.float32)
         + b_ref[...])
    mu = jnp.mean(z, axis=-1, keepdims=True)
    var = jnp.mean((z - mu) * (z - mu), axis=-1, keepdims=True)
    ln = (z - mu) * lax.rsqrt(var + 1e-5) * g_ref[...] + bb_ref[...]
    h = jnp.maximum(ln, 0.0) + x
    feat = jnp.concatenate([h, qb_ref[...]], axis=1)
    hm = jnp.maximum(
        jnp.dot(feat, w1_ref[...], preferred_element_type=jnp.float32)
        + b1_ref[...], 0.0)
    sc = jnp.sum(hm * w2_ref[...], axis=1, keepdims=True) + b2_ref[0]
    out_ref[...] = sc


def _call_tc_head(s1t, agg2, qb, bmask, wx, wa, brow, grow, bbrow,
                  mlp_w1, mlp_b1row, w2row, b2):
    return pl.pallas_call(
        _tc_head,
        out_shape=jax.ShapeDtypeStruct((BS * NEG, 1), jnp.float32),
        in_specs=[pl.BlockSpec(memory_space=pltpu.SMEM)] + [pl.BlockSpec()] * 12,
        name="tc_head",
    )(b2, s1t, agg2, qb, bmask, wx, wa, brow, grow, bbrow, mlp_w1, mlp_b1row,
      w2row)


def kernel(relation_representations, batch, edge_index, edge_type,
           Wr0, W0, b0, g0, bb0, Wr1, W1, b1, g1, bb1,
           mlp_W1, mlp_b1, mlp_W2, mlp_b2):
    h0 = batch[:, 0, 0].astype(jnp.int32)
    t_flat = batch[:, :, 1].reshape(BS * NEG).astype(jnp.int32)
    r0 = batch[:, 0, 2].astype(jnp.int32)
    src = edge_index[0]
    dst = edge_index[1]

    rel_flat = relation_representations.reshape(BS * R, D)
    rel1f, q8 = _call_tc_proj(rel_flat, Wr1, r0)
    query = q8[:BS]
    rel1 = rel1f.reshape(BS, R, D)

    h0p = jnp.zeros((16,), jnp.int32).at[:BS].set(h0)
    agg1 = _call_sc_layer1(src, edge_type, dst, h0p, relation_representations)

    state1 = _call_tc_dense1(
        agg1, q8, h0, Wr0, W0[:D], W0[D:], b0.reshape(1, D), g0.reshape(1, D),
        bb0.reshape(1, D))

    agg2, s1t = _call_sc_layer2(dst, src, edge_type, t_flat, rel1, state1)

    qb = jnp.repeat(query, NEG, axis=0)
    bmask = (t_flat == jnp.repeat(h0, NEG)).astype(jnp.float32)[:, None]
    score = _call_tc_head(
        s1t.reshape(BS * NEG, D), agg2.reshape(BS * NEG, D), qb, bmask,
        W1[:D], W1[D:], b1.reshape(1, D), g1.reshape(1, D), bb1.reshape(1, D),
        mlp_W1, mlp_b1.reshape(1, 2 * D), mlp_W2.reshape(1, 2 * D), mlp_b2)
    return score.reshape(BS, NEG)
